# baseline (speedup 1.0000x reference)
import jax
import jax.numpy as jnp
from jax import lax
from jax.experimental import pallas as pl
from jax.experimental.pallas import tpu as pltpu
from jax.experimental.pallas import tpu_sc as plsc

N = 10000
NP = 10240
E = 160000
EP = 160000
D = 256
Q = 64
NC = 2
NS = 16
K = 80
NB = 5
EPT = EP // NS
CH = EPT // K
RPT = NP // NS
ZR = 128


def _sc_body(feats4, col_h, row_h, s_out, deg16_out,
             col_v, row_v, zrow_v, degl_v, accum_s, *ring):
  c = lax.axis_index("c")
  s = lax.axis_index("s")
  zeros16 = jnp.zeros((16,), jnp.float32)
  ones16 = jnp.ones((16,), jnp.float32)

  def fill_zrow(i, _):
    r = i // (Q // 16)
    o = (i % (Q // 16)) * 16
    zrow_v[r, pl.ds(o, 16)] = zeros16
    return 0
  lax.fori_loop(0, ZR * (Q // 16), fill_zrow, 0)

  def fill_degl(i, _):
    degl_v[pl.ds(i * 16, 16)] = zeros16
    return 0
  lax.fori_loop(0, NP // 16, fill_degl, 0)

  pltpu.sync_copy(col_h.at[s], col_v)
  pltpu.sync_copy(row_h.at[s], row_v)

  def zero_accum():
    for j in range(RPT // ZR):
      pltpu.sync_copy(zrow_v, accum_s.at[pl.ds(s * RPT + j * ZR, ZR)])

  bufs = ring[:NB]
  binb = ring[NB:2 * NB]
  gsems = ring[2 * NB:3 * NB]
  ssems = ring[3 * NB:4 * NB]

  def prime(table):
    for k in range(NB):
      pltpu.async_copy(table.at[col_v.at[k]], binb[k], gsems[k])

  def convert(k):
    def conv_r(r, _):
      for rr in range(2):
        for o in range(Q // 32):
          x = plsc.bitcast(binb[k][2 * r + rr, pl.ds(32 * o, 32)], jnp.int32)
          ev = plsc.bitcast(x << 16, jnp.float32)
          od = plsc.bitcast(x & jnp.int32(-65536), jnp.float32)
          bufs[k][2 * r + rr, pl.ds(32 * o, 16)] = ev
          bufs[k][2 * r + rr, pl.ds(32 * o + 16, 16)] = od
      return 0
    lax.fori_loop(0, K // 2, conv_r, 0)

  def sweep(table, with_deg):
    def hist(g):
      if with_deg:
        for j in range(K // 16):
          idx16 = row_v[g, pl.ds(j * 16, 16)]
          plsc.addupdate_scatter(degl_v, [idx16], ones16)

    def body(t4, _):
      t = NB * t4
      for k in range(NB):
        g = t + k
        pltpu.make_async_copy(table.at[col_v.at[g]], binb[k], gsems[k]).wait()
        convert(k)
        pltpu.async_copy(bufs[k], accum_s.at[row_v.at[g]], ssems[k], add=True)
        hist(g)
        pltpu.make_async_copy(bufs[k], accum_s.at[row_v.at[g]], ssems[k]).wait()
        pltpu.async_copy(table.at[col_v.at[g + NB]], binb[k], gsems[k])
      return 0
    lax.fori_loop(0, (CH - NB) // NB, body, 0)
    for k in range(NB):
      g = CH - NB + k
      pltpu.make_async_copy(table.at[col_v.at[g]], binb[k], gsems[k]).wait()
      convert(k)
      pltpu.sync_copy(bufs[k], accum_s.at[row_v.at[g]], add=True)
      hist(g)

  def copy_out(cc, q):
    h = q % 2
    for j in range(RPT // ZR):
      sl = pl.ds(s * RPT + j * ZR, ZR)
      pltpu.sync_copy(accum_s.at[sl], s_out.at[cc, sl, pl.ds(h * Q, Q)])

  def both_passes(cc, qa, qb, with_deg):
    ta = feats4.at[pl.ds(qa, 4 * N - 3)]
    tb = feats4.at[pl.ds(qb, 4 * N - 3)]
    zero_accum()
    prime(ta)
    plsc.subcore_barrier()
    sweep(ta, with_deg)
    plsc.subcore_barrier()
    prime(tb)
    copy_out(cc, qa)
    zero_accum()
    plsc.subcore_barrier()
    sweep(tb, False)
    plsc.subcore_barrier()
    copy_out(cc, qb)

  @pl.when(c == 0)
  def _():
    both_passes(0, 0, 1, True)

  @pl.when(c == 1)
  def _():
    both_passes(1, 2, 3, False)

  @pl.when(c == 0)
  def _():
    pltpu.sync_copy(degl_v, deg16_out.at[s])


_sc_call = pl.kernel(
    _sc_body,
    out_type=(
        jax.ShapeDtypeStruct((NC, NP, 2 * Q), jnp.float32),
        jax.ShapeDtypeStruct((NS, NP), jnp.float32),
    ),
    mesh=plsc.VectorSubcoreMesh(core_axis_name="c", subcore_axis_name="s"),
    compiler_params=pltpu.CompilerParams(
        needs_layout_passes=False, use_tc_tiling_on_sc=False),
    scratch_types=[
        pltpu.VMEM((CH, K), jnp.int32),
        pltpu.VMEM((CH, K), jnp.int32),
        pltpu.VMEM((ZR, Q), jnp.float32),
        pltpu.VMEM((NP,), jnp.float32),
        pltpu.VMEM_SHARED((NP, Q), jnp.float32),
    ] + [pltpu.VMEM((K, Q), jnp.float32)] * NB
      + [pltpu.VMEM((K, Q), jnp.bfloat16)] * NB
      + [pltpu.SemaphoreType.DMA] * (2 * NB),
)


def _tc_body(sl_ref, sr_ref, deg_ref, wtp_ref, wt_ref, b_ref, out_ref):
  wtp = wtp_ref[...]
  wt = wt_ref[...]
  a = jnp.dot(sl_ref[0], wtp[:2 * Q, :], preferred_element_type=jnp.float32)
  a = a + jnp.dot(sr_ref[0], wtp[2 * Q:, :], preferred_element_type=jnp.float32)
  b16 = jnp.broadcast_to(b_ref[...], (NS, D))
  a = a + jnp.dot(deg_ref[...], b16, preferred_element_type=jnp.float32)
  out_ref[...] = jnp.dot(a, wt, preferred_element_type=jnp.float32) + b_ref[...]


_TC_R = 1000


def _tc_call(s2, deg16, wtp, wt, b2):
  return pl.pallas_call(
      _tc_body,
      grid=(N // _TC_R,),
      in_specs=[
          pl.BlockSpec((1, _TC_R, 2 * Q), lambda i: (0, i, 0)),
          pl.BlockSpec((1, _TC_R, 2 * Q), lambda i: (1, i, 0)),
          pl.BlockSpec((_TC_R, NS), lambda i: (i, 0)),
          pl.BlockSpec((D, D), lambda i: (0, 0)),
          pl.BlockSpec((D, D), lambda i: (0, 0)),
          pl.BlockSpec((1, D), lambda i: (0, 0)),
      ],
      out_specs=pl.BlockSpec((_TC_R, D), lambda i: (i, 0)),
      out_shape=jax.ShapeDtypeStruct((N, D), jnp.float32),
  )(s2, s2, deg16, wtp, wt, b2)


@jax.jit
def kernel(features, edge_index, W, b):
  row = edge_index[0].astype(jnp.int32)
  col = edge_index[1].astype(jnp.int32)
  col_h = (col * 4).reshape(NS, CH, K)
  row_h = row.reshape(NS, CH, K)
  feats4 = features.astype(jnp.bfloat16).reshape(4 * N, Q)
  s2, deg16 = _sc_call(feats4, col_h, row_h)
  deg16 = deg16.T
  perm = []
  for g in range(D):
    q, d = g // Q, g % Q
    o, m = d // 32, d % 32
    c = 32 * o + (2 * m if m < 16 else 2 * (m - 16) + 1)
    perm.append(q * Q + c)
  wt = W.T
  wtp = wt[jnp.array(perm), :]
  b2 = b.reshape(1, D)
  return _tc_call(s2, deg16, wtp, wt, b2)
